# baseline (speedup 1.0000x reference)
import jax
import jax.numpy as jnp
import numpy as np
from jax.experimental import pallas as pl
from jax.experimental.pallas import tpu as pltpu

N = 10000
E = 160000
D_IN = 256
H = 8
DH = 32
D_OUT = H * DH

_ROW_BLOCK = 1000


def _proj_kernel(x_ref, w_ref, a_ref, xp_ref, s_ref):
    xp = jnp.dot(x_ref[...], w_ref[...], preferred_element_type=jnp.float32)
    xp_ref[...] = xp
    s_ref[...] = jnp.dot(xp, a_ref[...], preferred_element_type=jnp.float32)


def _project(x, W, A):
    grid = (N // _ROW_BLOCK,)
    return pl.pallas_call(
        _proj_kernel,
        grid=grid,
        in_specs=[
            pl.BlockSpec((_ROW_BLOCK, D_IN), lambda i: (i, 0)),
            pl.BlockSpec((D_IN, D_OUT), lambda i: (0, 0)),
            pl.BlockSpec((D_OUT, 16), lambda i: (0, 0)),
        ],
        out_specs=[
            pl.BlockSpec((_ROW_BLOCK, D_OUT), lambda i: (i, 0)),
            pl.BlockSpec((_ROW_BLOCK, 16), lambda i: (i, 0)),
        ],
        out_shape=[
            jax.ShapeDtypeStruct((N, D_OUT), jnp.float32),
            jax.ShapeDtypeStruct((N, 16), jnp.float32),
        ],
    )(x, W, A)


def kernel(x, edge_indices, W, src_attn, dst_attn):
    eye = jnp.eye(H, dtype=jnp.float32)
    a_src = (src_attn[0][:, :, None] * eye[:, None, :]).reshape(D_OUT, H)
    a_dst = (dst_attn[0][:, :, None] * eye[:, None, :]).reshape(D_OUT, H)
    A = jnp.concatenate([a_src, a_dst], axis=1)

    xp_flat, s = _project(x, W, A)
    src_s = s[:, :H]
    dst_s = s[:, H:]

    row = edge_indices[0]
    col = edge_indices[1]
    e = src_s[row] + dst_s[col]
    e = jnp.where(e >= 0, e, 0.2 * e)
    m = jax.ops.segment_max(e, row, num_segments=N)
    ex = jnp.exp(e - m[row])
    denom = jax.ops.segment_sum(ex, row, num_segments=N)
    w = ex / denom[row]
    xp = xp_flat.reshape(N, H, DH)
    msgs = w[:, :, None] * xp[col]
    out = jax.ops.segment_sum(msgs, row, num_segments=N)
    return out.reshape(N, D_OUT)


# speedup vs baseline: 37.9699x; 37.9699x over previous
import functools

import jax
import jax.numpy as jnp
from jax import lax
from jax.experimental import pallas as pl
from jax.experimental.pallas import tpu as pltpu
from jax.experimental.pallas import tpu_sc as plsc

N = 10000
E = 160000
D_IN = 256
H = 8
DH = 32
D_OUT = H * DH
QW = 64

NC = 2
NS = 16
LN = 16

BLK = 16
NB = 628
EPT = NB * BLK
E_PAD = EPT * NS
NBUF = 4
ACCW = 80
NCHUNK = N // BLK

_ROW_BLOCK = 1000


def _proj_body(x_ref, w_ref, a_ref, xp4_ref, ssrc_ref, sdst_ref, smax_ref):
    i = pl.program_id(0)
    xp = jnp.dot(x_ref[...], w_ref[...], preferred_element_type=jnp.float32)
    for q in range(4):
        xp4_ref[q] = xp[:, QW * q:QW * (q + 1)]
    s = jnp.dot(xp, a_ref[...], preferred_element_type=jnp.float32)
    z = jnp.zeros((s.shape[0], 12), jnp.float32)
    ssrc_ref[0] = jnp.concatenate([s[:, 0:4], z], axis=1)
    sdst_ref[0] = jnp.concatenate([s[:, 4:8], z], axis=1)
    ssrc_ref[1] = jnp.concatenate([s[:, 8:12], z], axis=1)
    sdst_ref[1] = jnp.concatenate([s[:, 12:16], z], axis=1)
    bm = jnp.max(s, axis=0, keepdims=True)
    @pl.when(i == 0)
    def _():
        smax_ref[...] = jnp.full((1, 16), -1e30, jnp.float32)
    smax_ref[...] = jnp.maximum(smax_ref[...], bm)


def _project(x, W, A):
    return pl.pallas_call(
        _proj_body,
        grid=(N // _ROW_BLOCK,),
        in_specs=[
            pl.BlockSpec((_ROW_BLOCK, D_IN), lambda i: (i, 0)),
            pl.BlockSpec((D_IN, D_OUT), lambda i: (0, 0)),
            pl.BlockSpec((D_OUT, 16), lambda i: (0, 0)),
        ],
        out_specs=[
            pl.BlockSpec((4, _ROW_BLOCK, QW), lambda i: (0, i, 0)),
            pl.BlockSpec((2, _ROW_BLOCK, 16), lambda i: (0, i, 0)),
            pl.BlockSpec((2, _ROW_BLOCK, 16), lambda i: (0, i, 0)),
            pl.BlockSpec((1, 16), lambda i: (0, 0)),
        ],
        out_shape=[
            jax.ShapeDtypeStruct((4, N, QW), jnp.float32),
            jax.ShapeDtypeStruct((2, N, 16), jnp.float32),
            jax.ShapeDtypeStruct((2, N, 16), jnp.float32),
            jax.ShapeDtypeStruct((1, 16), jnp.float32),
        ],
    )(x, W, A)


def _sc_body(xpf_h, ssrc_h, sdst_h, smax_h, row3_h, colx3_h, cols3_h, out_h,
             rows_v, colx_v, cols_v, gx, gs, gd, mb, wv, smax_v, nbuf, obuf,
             acc, gxsem, gssem, gdsem, scsem):
    c = lax.axis_index("c")
    sid = lax.axis_index("s")
    i16 = lax.iota(jnp.int32, 16)
    cN = c * N
    cN1 = c * (N + 1)

    pltpu.sync_copy(row3_h.at[sid], rows_v)
    pltpu.sync_copy(colx3_h.at[sid], colx_v)
    pltpu.sync_copy(cols3_h.at[sid], cols_v)
    pltpu.sync_copy(smax_h.at[0], smax_v)

    ms = plsc.load_gather(smax_v, [i16 + 8 * c])
    md = plsc.load_gather(smax_v, [jnp.minimum(i16 + (8 * c + 4), 15)])
    mv = jnp.where(i16 < 4, jnp.maximum(ms + md, 0.0), 1e30)

    zero16 = jnp.zeros((16,), jnp.float32)
    ohf = [jnp.where(i16 == h, 1.0, 0.0).astype(jnp.float32) for h in range(4)]
    jsplat = [jnp.full((16,), j, jnp.int32) for j in range(BLK)]
    dsplat = [jnp.full((16,), QW + h, jnp.int32) for h in range(4)]

    def zero_acc():
        for r in range(BLK):
            for k in range(ACCW // LN):
                mb[0, r, pl.ds(LN * k, LN)] = zero16
        @pl.loop(0, 39)
        def _(t):
            pltpu.sync_copy(mb.at[0], acc.at[pl.ds((t * NS + sid) * BLK, BLK)])
        @pl.when(sid == 0)
        def _():
            pltpu.sync_copy(mb.at[0], acc.at[pl.ds((NCHUNK - 1) * BLK, BLK)])
        plsc.subcore_barrier()

    for s in range(2):
        zero_acc()
        qN = (2 * c + s) * N

        def gather_descs(blk, b):
            return (
                pltpu.make_async_copy(
                    xpf_h.at[colx_v[blk] + qN], gx.at[b], gxsem[b]),
                pltpu.make_async_copy(
                    ssrc_h.at[rows_v[blk] + cN], gs.at[b], gssem[b]),
                pltpu.make_async_copy(
                    sdst_h.at[cols_v[blk] + cN1], gd.at[b], gdsem[b]),
            )

        for b in range(NBUF):
            for d in gather_descs(b, b):
                d.start()

        @pl.loop(0, NB, step=NBUF)
        def _(i0):
            for b in range(NBUF):
                blk = i0 + b
                for d in gather_descs(blk, b):
                    d.wait()
                @pl.when(blk >= NBUF)
                def _():
                    pltpu.make_async_copy(
                        mb.at[b], acc.at[rows_v[blk - NBUF]], scsem[b]).wait()
                for j in range(BLK):
                    e = gs[b, j] + gd[b, j]
                    e = jnp.where(e >= 0.0, e, 0.2 * e)
                    ex = jnp.exp(e - mv)
                    mb[b, j, pl.ds(QW, 16)] = ex
                    for k in range(2):
                        ws = jnp.full((16,), jnp.sum(ex * ohf[2 * s + k]),
                                      jnp.float32)
                        for r2 in range(2):
                            off = k * 32 + r2 * LN
                            mb[b, j, pl.ds(off, LN)] = (
                                gx[b, j, pl.ds(off, LN)] * ws)
                pltpu.async_copy(mb.at[b], acc.at[rows_v[blk]], scsem[b],
                                 add=True)
                @pl.when(blk + NBUF < NB)
                def _():
                    for d in gather_descs(blk + NBUF, b):
                        d.start()

        for b in range(NBUF):
            pltpu.make_async_copy(
                mb.at[b], acc.at[rows_v[NB - NBUF + b]], scsem[b]).wait()
        plsc.subcore_barrier()

        def norm_chunk(g):
            r0 = g * BLK
            pltpu.sync_copy(acc.at[pl.ds(r0, BLK)], nbuf)
            for j in range(BLK):
                for k in range(2):
                    dn = plsc.load_gather(nbuf, [jsplat[j], dsplat[2 * s + k]])
                    rec = 1.0 / jnp.maximum(dn, 1e-30)
                    for r2 in range(2):
                        off = k * 32 + r2 * LN
                        obuf[j, pl.ds(off, LN)] = nbuf[j, pl.ds(off, LN)] * rec
            pltpu.sync_copy(obuf, out_h.at[pl.ds(qN + r0, BLK)])

        @pl.loop(0, 39)
        def _(t):
            norm_chunk(t * NS + sid)
        @pl.when(sid == 0)
        def _():
            norm_chunk(NCHUNK - 1)
        plsc.subcore_barrier()


def _sc_aggregate(xpf, ssrc_f, sdst_p, smax, row3, colx3, cols3):
    mesh = plsc.VectorSubcoreMesh(core_axis_name="c", subcore_axis_name="s")
    cp = pltpu.CompilerParams(
        needs_layout_passes=False, use_tc_tiling_on_sc=False)
    run = pl.kernel(
        _sc_body,
        compiler_params=cp,
        out_type=jax.ShapeDtypeStruct((4 * N, QW), jnp.float32),
        mesh=mesh,
        scratch_types=[
            pltpu.VMEM((NB, BLK), jnp.int32),
            pltpu.VMEM((NB, BLK), jnp.int32),
            pltpu.VMEM((NB, BLK), jnp.int32),
            pltpu.VMEM((NBUF, BLK, QW), jnp.float32),
            pltpu.VMEM((NBUF, BLK, 16), jnp.float32),
            pltpu.VMEM((NBUF, BLK, 16), jnp.float32),
            pltpu.VMEM((NBUF, BLK, ACCW), jnp.float32),
            pltpu.VMEM((16,), jnp.float32),
            pltpu.VMEM((16,), jnp.float32),
            pltpu.VMEM((BLK, ACCW), jnp.float32),
            pltpu.VMEM((BLK, QW), jnp.float32),
            pltpu.VMEM_SHARED((N, ACCW), jnp.float32),
            [pltpu.SemaphoreType.DMA] * NBUF,
            [pltpu.SemaphoreType.DMA] * NBUF,
            [pltpu.SemaphoreType.DMA] * NBUF,
            [pltpu.SemaphoreType.DMA] * NBUF,
        ],
    )
    return run(xpf, ssrc_f, sdst_p, smax, row3, colx3, cols3)


def kernel(x, edge_indices, W, src_attn, dst_attn):
    eye = jnp.eye(H, dtype=jnp.float32)
    asrc = (src_attn[0][:, :, None] * eye[:, None, :]).reshape(D_OUT, H)
    adst = (dst_attn[0][:, :, None] * eye[:, None, :]).reshape(D_OUT, H)
    A = jnp.concatenate(
        [asrc[:, 0:4], adst[:, 0:4], asrc[:, 4:8], adst[:, 4:8]], axis=1)

    xp4, ssrc, sdst, smax = _project(x, W, A)
    xpf = xp4.reshape(4 * N, QW)
    ssrc_f = ssrc.reshape(2 * N, 16)
    dummy = jnp.full((1, 16), -1e30, jnp.float32)
    sdst_p = jnp.concatenate([sdst[0], dummy, sdst[1], dummy], axis=0)

    row = edge_indices[0].astype(jnp.int32)
    col = edge_indices[1].astype(jnp.int32)
    padn = E_PAD - E
    zpad = jnp.zeros((padn,), jnp.int32)
    row3 = jnp.concatenate([row, zpad]).reshape(NS, NB, BLK)
    colx3 = jnp.concatenate([col, zpad]).reshape(NS, NB, BLK)
    cols3 = jnp.concatenate(
        [col, jnp.full((padn,), N, jnp.int32)]).reshape(NS, NB, BLK)

    out = _sc_aggregate(xpf, ssrc_f, sdst_p, smax, row3, colx3, cols3)
    o4 = out.reshape(4, N, QW)
    return jnp.concatenate([o4[0], o4[1], o4[2], o4[3]], axis=1)


# speedup vs baseline: 43.0526x; 1.1339x over previous
import functools

import jax
import jax.numpy as jnp
from jax import lax
from jax.experimental import pallas as pl
from jax.experimental.pallas import tpu as pltpu
from jax.experimental.pallas import tpu_sc as plsc

N = 10000
E = 160000
D_IN = 256
H = 8
DH = 32
D_OUT = H * DH
QW = 64
QROW = 16 + QW

NC = 2
NS = 16
LN = 16

BLK = 16
NB = 632
EPT = NB * BLK
E_PAD = EPT * NS
NBUF = 8
ACCW = 80
NCHUNK = N // BLK

_ROW_BLOCK = 1000
NPAD = N + _ROW_BLOCK


def _proj_body(x_ref, w_ref, a_ref, xpq_ref, ssrc_ref, smax_ref):
    i = pl.program_id(0)
    xp = jnp.dot(x_ref[...], w_ref[...], preferred_element_type=jnp.float32)
    s = jnp.dot(xp, a_ref[...], preferred_element_type=jnp.float32)
    z = jnp.zeros((s.shape[0], 12), jnp.float32)

    @pl.when(i <= 9)
    def _():
        for q in range(4):
            dst = jnp.concatenate([s[:, 8 * (q // 2) + 4:8 * (q // 2) + 8], z],
                                  axis=1)
            xpq_ref[q] = jnp.concatenate(
                [dst, xp[:, QW * q:QW * (q + 1)]], axis=1)
        ssrc_ref[0] = jnp.concatenate([s[:, 0:4], z], axis=1)
        ssrc_ref[1] = jnp.concatenate([s[:, 8:12], z], axis=1)

    @pl.when(i == 10)
    def _():
        dummy = jnp.concatenate(
            [jnp.full((_ROW_BLOCK, 16), -1e30, jnp.float32),
             jnp.zeros((_ROW_BLOCK, QW), jnp.float32)], axis=1)
        for q in range(4):
            xpq_ref[q] = dummy

    bm = jnp.max(s, axis=0, keepdims=True)
    @pl.when(i == 0)
    def _():
        smax_ref[...] = jnp.full((1, 16), -1e30, jnp.float32)
    smax_ref[...] = jnp.maximum(smax_ref[...], bm)


def _project(x, W, A):
    return pl.pallas_call(
        _proj_body,
        grid=(11,),
        in_specs=[
            pl.BlockSpec((_ROW_BLOCK, D_IN), lambda i: (jnp.minimum(i, 9), 0)),
            pl.BlockSpec((D_IN, D_OUT), lambda i: (0, 0)),
            pl.BlockSpec((D_OUT, 16), lambda i: (0, 0)),
        ],
        out_specs=[
            pl.BlockSpec((4, _ROW_BLOCK, QROW), lambda i: (0, i, 0)),
            pl.BlockSpec((2, _ROW_BLOCK, 16),
                         lambda i: (0, jnp.minimum(i, 9), 0)),
            pl.BlockSpec((1, 16), lambda i: (0, 0)),
        ],
        out_shape=[
            jax.ShapeDtypeStruct((4, NPAD, QROW), jnp.float32),
            jax.ShapeDtypeStruct((2, N, 16), jnp.float32),
            jax.ShapeDtypeStruct((1, 16), jnp.float32),
        ],
    )(x, W, A)


def _sc_body(xpq_h, ssrc_h, smax_h, row3_h, colx3_h, out_h,
             rows_v, colx_v, gx, gs, mb, smax_v, nbuf, obuf,
             acc, gxsem, gssem, scsem):
    c = lax.axis_index("c")
    sid = lax.axis_index("s")
    i16 = lax.iota(jnp.int32, 16)
    cN = c * N

    pltpu.sync_copy(row3_h.at[sid], rows_v)
    pltpu.sync_copy(colx3_h.at[sid], colx_v)
    pltpu.sync_copy(smax_h.at[0], smax_v)

    ms = plsc.load_gather(smax_v, [i16 + 8 * c])
    md = plsc.load_gather(smax_v, [jnp.minimum(i16 + (8 * c + 4), 15)])
    mv = jnp.where(i16 < 4, jnp.maximum(ms + md, 0.0), 1e30)

    zero16 = jnp.zeros((16,), jnp.float32)
    ohf = [jnp.where(i16 == h, 1.0, 0.0).astype(jnp.float32) for h in range(4)]
    jsplat = [jnp.full((16,), j, jnp.int32) for j in range(BLK)]
    dsplat = [jnp.full((16,), QW + h, jnp.int32) for h in range(4)]

    def zero_acc():
        for r in range(BLK):
            for k in range(ACCW // LN):
                mb[0, r, pl.ds(LN * k, LN)] = zero16
        @pl.loop(0, 39)
        def _(t):
            pltpu.sync_copy(mb.at[0], acc.at[pl.ds((t * NS + sid) * BLK, BLK)])
        @pl.when(sid == 0)
        def _():
            pltpu.sync_copy(mb.at[0], acc.at[pl.ds((NCHUNK - 1) * BLK, BLK)])
        plsc.subcore_barrier()

    for s in range(2):
        zero_acc()
        q = 2 * c + s
        qbase = q * NPAD

        def gather_descs(blk, b):
            return (
                pltpu.make_async_copy(
                    xpq_h.at[colx_v[blk] + qbase], gx.at[b], gxsem[b]),
                pltpu.make_async_copy(
                    ssrc_h.at[rows_v[blk] + cN], gs.at[b], gssem[b]),
            )

        for b in range(NBUF):
            for d in gather_descs(b, b):
                d.start()

        @pl.loop(0, NB, step=NBUF)
        def _(i0):
            for b in range(NBUF):
                blk = i0 + b
                for d in gather_descs(blk, b):
                    d.wait()
                @pl.when(blk >= NBUF)
                def _():
                    pltpu.make_async_copy(
                        mb.at[b], acc.at[rows_v[blk - NBUF]], scsem[b]).wait()
                for j in range(BLK):
                    e = gs[b, j] + gx[b, j, pl.ds(0, 16)]
                    e = jnp.where(e >= 0.0, e, 0.2 * e)
                    ex = jnp.exp(e - mv)
                    mb[b, j, pl.ds(QW, 16)] = ex
                    for k in range(2):
                        ws = jnp.full((16,), jnp.sum(ex * ohf[2 * s + k]),
                                      jnp.float32)
                        for r2 in range(2):
                            off = k * 32 + r2 * LN
                            mb[b, j, pl.ds(off, LN)] = (
                                gx[b, j, pl.ds(16 + off, LN)] * ws)
                pltpu.async_copy(mb.at[b], acc.at[rows_v[blk]], scsem[b],
                                 add=True)
                @pl.when(blk + NBUF < NB)
                def _():
                    for d in gather_descs(blk + NBUF, b):
                        d.start()

        for b in range(NBUF):
            pltpu.make_async_copy(
                mb.at[b], acc.at[rows_v[NB - NBUF + b]], scsem[b]).wait()
        plsc.subcore_barrier()

        def norm_chunk(g):
            r0 = g * BLK
            pltpu.sync_copy(acc.at[pl.ds(r0, BLK)], nbuf)
            for j in range(BLK):
                for k in range(2):
                    dn = plsc.load_gather(nbuf, [jsplat[j], dsplat[2 * s + k]])
                    rec = 1.0 / jnp.maximum(dn, 1e-30)
                    for r2 in range(2):
                        off = k * 32 + r2 * LN
                        obuf[j, pl.ds(off, LN)] = nbuf[j, pl.ds(off, LN)] * rec
            pltpu.sync_copy(obuf, out_h.at[pl.ds(r0, BLK), pl.ds(QW * q, QW)])

        @pl.loop(0, 39)
        def _(t):
            norm_chunk(t * NS + sid)
        @pl.when(sid == 0)
        def _():
            norm_chunk(NCHUNK - 1)
        plsc.subcore_barrier()


def _sc_aggregate(xpq, ssrc_f, smax, row3, colx3):
    mesh = plsc.VectorSubcoreMesh(core_axis_name="c", subcore_axis_name="s")
    cp = pltpu.CompilerParams(
        needs_layout_passes=False, use_tc_tiling_on_sc=False)
    run = pl.kernel(
        _sc_body,
        compiler_params=cp,
        out_type=jax.ShapeDtypeStruct((N, D_OUT), jnp.float32),
        mesh=mesh,
        scratch_types=[
            pltpu.VMEM((NB, BLK), jnp.int32),
            pltpu.VMEM((NB, BLK), jnp.int32),
            pltpu.VMEM((NBUF, BLK, QROW), jnp.float32),
            pltpu.VMEM((NBUF, BLK, 16), jnp.float32),
            pltpu.VMEM((NBUF, BLK, ACCW), jnp.float32),
            pltpu.VMEM((16,), jnp.float32),
            pltpu.VMEM((BLK, ACCW), jnp.float32),
            pltpu.VMEM((BLK, QW), jnp.float32),
            pltpu.VMEM_SHARED((N, ACCW), jnp.float32),
            [pltpu.SemaphoreType.DMA] * NBUF,
            [pltpu.SemaphoreType.DMA] * NBUF,
            [pltpu.SemaphoreType.DMA] * NBUF,
        ],
    )
    return run(xpq, ssrc_f, smax, row3, colx3)


def kernel(x, edge_indices, W, src_attn, dst_attn):
    eye = jnp.eye(H, dtype=jnp.float32)
    asrc = (src_attn[0][:, :, None] * eye[:, None, :]).reshape(D_OUT, H)
    adst = (dst_attn[0][:, :, None] * eye[:, None, :]).reshape(D_OUT, H)
    A = jnp.concatenate(
        [asrc[:, 0:4], adst[:, 0:4], asrc[:, 4:8], adst[:, 4:8]], axis=1)

    xpq, ssrc, smax = _project(x, W, A)
    xpq_f = xpq.reshape(4 * NPAD, QROW)
    ssrc_f = ssrc.reshape(2 * N, 16)

    row = edge_indices[0].astype(jnp.int32)
    col = edge_indices[1].astype(jnp.int32)
    padn = E_PAD - E
    row3 = jnp.concatenate(
        [row, jnp.zeros((padn,), jnp.int32)]).reshape(NS, NB, BLK)
    colx3 = jnp.concatenate(
        [col, jnp.full((padn,), N, jnp.int32)]).reshape(NS, NB, BLK)

    return _sc_aggregate(xpq_f, ssrc_f, smax, row3, colx3)


# speedup vs baseline: 43.1301x; 1.0018x over previous
import functools

import jax
import jax.numpy as jnp
from jax import lax
from jax.experimental import pallas as pl
from jax.experimental.pallas import tpu as pltpu
from jax.experimental.pallas import tpu_sc as plsc

N = 10000
E = 160000
D_IN = 256
H = 8
DH = 32
D_OUT = H * DH
QW = 64
QROW = 16 + QW

NC = 2
NS = 16
LN = 16

BLK = 16
NB = 632
EPT = NB * BLK
E_PAD = EPT * NS
NBUF = 8
ACCW = 80
NCHUNK = N // BLK

_ROW_BLOCK = 1000
NPAD = N + _ROW_BLOCK


def _proj_body(x_ref, w_ref, a_ref, xpq_ref, ssrc_ref, smax_ref):
    i = pl.program_id(0)
    xp = jnp.dot(x_ref[...], w_ref[...], preferred_element_type=jnp.float32)
    s = jnp.dot(xp, a_ref[...], preferred_element_type=jnp.float32)
    z = jnp.zeros((s.shape[0], 12), jnp.float32)

    @pl.when(i <= 9)
    def _():
        for q in range(4):
            dst = jnp.concatenate([s[:, 8 * (q // 2) + 4:8 * (q // 2) + 8], z],
                                  axis=1)
            xpq_ref[q] = jnp.concatenate(
                [dst, xp[:, QW * q:QW * (q + 1)]], axis=1)
        ssrc_ref[0] = jnp.concatenate([s[:, 0:4], z], axis=1)
        ssrc_ref[1] = jnp.concatenate([s[:, 8:12], z], axis=1)

    @pl.when(i == 10)
    def _():
        dummy = jnp.concatenate(
            [jnp.full((_ROW_BLOCK, 16), -1e30, jnp.float32),
             jnp.zeros((_ROW_BLOCK, QW), jnp.float32)], axis=1)
        for q in range(4):
            xpq_ref[q] = dummy

    bm = jnp.max(s, axis=0, keepdims=True)
    @pl.when(i == 0)
    def _():
        smax_ref[...] = jnp.full((1, 16), -1e30, jnp.float32)
    smax_ref[...] = jnp.maximum(smax_ref[...], bm)


def _project(x, W, A):
    return pl.pallas_call(
        _proj_body,
        grid=(11,),
        in_specs=[
            pl.BlockSpec((_ROW_BLOCK, D_IN), lambda i: (jnp.minimum(i, 9), 0)),
            pl.BlockSpec((D_IN, D_OUT), lambda i: (0, 0)),
            pl.BlockSpec((D_OUT, 16), lambda i: (0, 0)),
        ],
        out_specs=[
            pl.BlockSpec((4, _ROW_BLOCK, QROW), lambda i: (0, i, 0)),
            pl.BlockSpec((2, _ROW_BLOCK, 16),
                         lambda i: (0, jnp.minimum(i, 9), 0)),
            pl.BlockSpec((1, 16), lambda i: (0, 0)),
        ],
        out_shape=[
            jax.ShapeDtypeStruct((4, NPAD, QROW), jnp.float32),
            jax.ShapeDtypeStruct((2, N, 16), jnp.float32),
            jax.ShapeDtypeStruct((1, 16), jnp.float32),
        ],
    )(x, W, A)


def _sc_body(xpq_h, ssrc_h, smax_h, row3_h, colx3_h, out_h,
             rows_v, colx_v, gx, gs, mb, smax_v, nbuf, obuf,
             acc, gxsem, gssem, scsem):
    c = lax.axis_index("c")
    sid = lax.axis_index("s")
    i16 = lax.iota(jnp.int32, 16)
    cN = c * N

    pltpu.sync_copy(row3_h.at[sid], rows_v)
    pltpu.sync_copy(colx3_h.at[sid], colx_v)
    pltpu.sync_copy(smax_h.at[0], smax_v)

    ms = plsc.load_gather(smax_v, [i16 + 8 * c])
    md = plsc.load_gather(smax_v, [jnp.minimum(i16 + (8 * c + 4), 15)])
    mv = jnp.where(i16 < 4, jnp.maximum(ms + md, 0.0), 1e30)

    zero16 = jnp.zeros((16,), jnp.float32)
    ohf = [jnp.where(i16 == h, 1.0, 0.0).astype(jnp.float32) for h in range(4)]
    jsplat = [jnp.full((16,), j, jnp.int32) for j in range(BLK)]
    dsplat = [jnp.full((16,), QW + h, jnp.int32) for h in range(4)]

    def zero_acc():
        for r in range(BLK):
            for k in range(ACCW // LN):
                mb[0, r, pl.ds(LN * k, LN)] = zero16
        @pl.loop(0, 39)
        def _(t):
            pltpu.sync_copy(mb.at[0], acc.at[pl.ds((t * NS + sid) * BLK, BLK)])
        @pl.when(sid == 0)
        def _():
            pltpu.sync_copy(mb.at[0], acc.at[pl.ds((NCHUNK - 1) * BLK, BLK)])
        plsc.subcore_barrier()

    for s in range(2):
        zero_acc()
        q = 2 * c + s
        qbase = q * NPAD

        def gather_descs(blk, b):
            return (
                pltpu.make_async_copy(
                    xpq_h.at[colx_v[blk] + qbase], gx.at[b], gxsem[b]),
                pltpu.make_async_copy(
                    ssrc_h.at[rows_v[blk] + cN], gs.at[b], gssem[b]),
            )

        for b in range(NBUF):
            for d in gather_descs(b, b):
                d.start()

        @pl.loop(0, NB, step=NBUF)
        def _(i0):
            for b in range(NBUF):
                blk = i0 + b
                for d in gather_descs(blk, b):
                    d.wait()
                @pl.when(blk >= NBUF)
                def _():
                    pltpu.make_async_copy(
                        mb.at[b], acc.at[rows_v[blk - NBUF]], scsem[b]).wait()
                for j in range(BLK):
                    e = gs[b, j] + gx[b, j, pl.ds(0, 16)]
                    e = jnp.maximum(e, 0.2 * e)
                    ex = jnp.exp(e - mv)
                    mb[b, j, pl.ds(QW, 16)] = ex
                    for k in range(2):
                        ws = jnp.full((16,), jnp.sum(ex * ohf[2 * s + k]),
                                      jnp.float32)
                        for r2 in range(2):
                            off = k * 32 + r2 * LN
                            mb[b, j, pl.ds(off, LN)] = (
                                gx[b, j, pl.ds(16 + off, LN)] * ws)
                pltpu.async_copy(mb.at[b], acc.at[rows_v[blk]], scsem[b],
                                 add=True)
                @pl.when(blk + NBUF < NB)
                def _():
                    for d in gather_descs(blk + NBUF, b):
                        d.start()

        for b in range(NBUF):
            pltpu.make_async_copy(
                mb.at[b], acc.at[rows_v[NB - NBUF + b]], scsem[b]).wait()
        plsc.subcore_barrier()

        def norm_chunk(g):
            r0 = g * BLK
            pltpu.sync_copy(acc.at[pl.ds(r0, BLK)], nbuf)
            for j in range(BLK):
                for k in range(2):
                    dn = plsc.load_gather(nbuf, [jsplat[j], dsplat[2 * s + k]])
                    rec = 1.0 / jnp.maximum(dn, 1e-30)
                    for r2 in range(2):
                        off = k * 32 + r2 * LN
                        obuf[j, pl.ds(off, LN)] = nbuf[j, pl.ds(off, LN)] * rec
            pltpu.sync_copy(obuf, out_h.at[pl.ds(r0, BLK), pl.ds(QW * q, QW)])

        @pl.loop(0, 39)
        def _(t):
            norm_chunk(t * NS + sid)
        @pl.when(sid == 0)
        def _():
            norm_chunk(NCHUNK - 1)
        plsc.subcore_barrier()


def _sc_aggregate(xpq, ssrc_f, smax, row3, colx3):
    mesh = plsc.VectorSubcoreMesh(core_axis_name="c", subcore_axis_name="s")
    cp = pltpu.CompilerParams(
        needs_layout_passes=False, use_tc_tiling_on_sc=False)
    run = pl.kernel(
        _sc_body,
        compiler_params=cp,
        out_type=jax.ShapeDtypeStruct((N, D_OUT), jnp.float32),
        mesh=mesh,
        scratch_types=[
            pltpu.VMEM((NB, BLK), jnp.int32),
            pltpu.VMEM((NB, BLK), jnp.int32),
            pltpu.VMEM((NBUF, BLK, QROW), jnp.float32),
            pltpu.VMEM((NBUF, BLK, 16), jnp.float32),
            pltpu.VMEM((NBUF, BLK, ACCW), jnp.float32),
            pltpu.VMEM((16,), jnp.float32),
            pltpu.VMEM((BLK, ACCW), jnp.float32),
            pltpu.VMEM((BLK, QW), jnp.float32),
            pltpu.VMEM_SHARED((N, ACCW), jnp.float32),
            [pltpu.SemaphoreType.DMA] * NBUF,
            [pltpu.SemaphoreType.DMA] * NBUF,
            [pltpu.SemaphoreType.DMA] * NBUF,
        ],
    )
    return run(xpq, ssrc_f, smax, row3, colx3)


def kernel(x, edge_indices, W, src_attn, dst_attn):
    eye = jnp.eye(H, dtype=jnp.float32)
    asrc = (src_attn[0][:, :, None] * eye[:, None, :]).reshape(D_OUT, H)
    adst = (dst_attn[0][:, :, None] * eye[:, None, :]).reshape(D_OUT, H)
    A = jnp.concatenate(
        [asrc[:, 0:4], adst[:, 0:4], asrc[:, 4:8], adst[:, 4:8]], axis=1)

    xpq, ssrc, smax = _project(x, W, A)
    xpq_f = xpq.reshape(4 * NPAD, QROW)
    ssrc_f = ssrc.reshape(2 * N, 16)

    row = edge_indices[0].astype(jnp.int32)
    col = edge_indices[1].astype(jnp.int32)
    padn = E_PAD - E
    row3 = jnp.concatenate(
        [row, jnp.zeros((padn,), jnp.int32)]).reshape(NS, NB, BLK)
    colx3 = jnp.concatenate(
        [col, jnp.full((padn,), N, jnp.int32)]).reshape(NS, NB, BLK)

    return _sc_aggregate(xpq_f, ssrc_f, smax, row3, colx3)


# speedup vs baseline: 52.1359x; 1.2088x over previous
import functools

import jax
import jax.numpy as jnp
from jax import lax
from jax.experimental import pallas as pl
from jax.experimental.pallas import tpu as pltpu
from jax.experimental.pallas import tpu_sc as plsc

N = 10000
E = 160000
D_IN = 256
H = 8
DH = 32
D_OUT = H * DH
QW = 64
QROW = 16 + QW // 2

NC = 2
NS = 16
LN = 16

BLK = 16
NB = 632
EPT = NB * BLK
E_PAD = EPT * NS
NBUF = 8
ACCW = 80
NCHUNK = N // BLK

_ROW_BLOCK = 1000
NPAD = N + _ROW_BLOCK


def _proj_body(x_ref, w_ref, a_ref, xpq_ref, ssrc_ref, smax_ref):
    i = pl.program_id(0)
    xp = jnp.dot(x_ref[...], w_ref[...], preferred_element_type=jnp.float32)
    s = jnp.dot(xp, a_ref[...], preferred_element_type=jnp.float32)
    z = jnp.zeros((s.shape[0], 12), jnp.float32)

    xpb = jax.lax.bitcast_convert_type(
        xp.astype(jnp.bfloat16), jnp.int16).astype(jnp.int32)
    words = []
    for k in range(8):
        lo = xpb[:, 32 * k:32 * k + 16] & 0xFFFF
        hi = xpb[:, 32 * k + 16:32 * k + 32] << 16
        words.append(lo | hi)
    packed = jax.lax.bitcast_convert_type(
        jnp.concatenate(words, axis=1), jnp.float32)

    @pl.when(i <= 9)
    def _():
        for q in range(4):
            dst = jnp.concatenate([s[:, 8 * (q // 2) + 4:8 * (q // 2) + 8], z],
                                  axis=1)
            xpq_ref[q] = jnp.concatenate(
                [dst, packed[:, 32 * q:32 * (q + 1)]], axis=1)
        ssrc_ref[0] = jnp.concatenate([s[:, 0:4], z], axis=1)
        ssrc_ref[1] = jnp.concatenate([s[:, 8:12], z], axis=1)

    @pl.when(i == 10)
    def _():
        dummy = jnp.concatenate(
            [jnp.full((_ROW_BLOCK, 16), -1e30, jnp.float32),
             jnp.zeros((_ROW_BLOCK, QW // 2), jnp.float32)], axis=1)
        for q in range(4):
            xpq_ref[q] = dummy

    bm = jnp.max(s, axis=0, keepdims=True)
    @pl.when(i == 0)
    def _():
        smax_ref[...] = jnp.full((1, 16), -1e30, jnp.float32)
    smax_ref[...] = jnp.maximum(smax_ref[...], bm)


def _project(x, W, A):
    return pl.pallas_call(
        _proj_body,
        grid=(11,),
        in_specs=[
            pl.BlockSpec((_ROW_BLOCK, D_IN), lambda i: (jnp.minimum(i, 9), 0)),
            pl.BlockSpec((D_IN, D_OUT), lambda i: (0, 0)),
            pl.BlockSpec((D_OUT, 16), lambda i: (0, 0)),
        ],
        out_specs=[
            pl.BlockSpec((4, _ROW_BLOCK, QROW), lambda i: (0, i, 0)),
            pl.BlockSpec((2, _ROW_BLOCK, 16),
                         lambda i: (0, jnp.minimum(i, 9), 0)),
            pl.BlockSpec((1, 16), lambda i: (0, 0)),
        ],
        out_shape=[
            jax.ShapeDtypeStruct((4, NPAD, QROW), jnp.float32),
            jax.ShapeDtypeStruct((2, N, 16), jnp.float32),
            jax.ShapeDtypeStruct((1, 16), jnp.float32),
        ],
    )(x, W, A)


def _sc_body(xpq_h, ssrc_h, smax_h, row3_h, colx3_h, out_h,
             rows_v, colx_v, gx, gs, mb, smax_v, nbuf, obuf,
             acc, gxsem, gssem, scsem):
    c = lax.axis_index("c")
    sid = lax.axis_index("s")
    i16 = lax.iota(jnp.int32, 16)
    cN = c * N

    pltpu.sync_copy(row3_h.at[sid], rows_v)
    pltpu.sync_copy(colx3_h.at[sid], colx_v)
    pltpu.sync_copy(smax_h.at[0], smax_v)

    ms = plsc.load_gather(smax_v, [i16 + 8 * c])
    md = plsc.load_gather(smax_v, [jnp.minimum(i16 + (8 * c + 4), 15)])
    mv = jnp.where(i16 < 4, jnp.maximum(ms + md, 0.0), 1e30)

    zero16 = jnp.zeros((16,), jnp.float32)
    ohf = [jnp.where(i16 == h, 1.0, 0.0).astype(jnp.float32) for h in range(4)]
    jsplat = [jnp.full((16,), j, jnp.int32) for j in range(BLK)]
    dsplat = [jnp.full((16,), QW + h, jnp.int32) for h in range(4)]

    def zero_acc():
        for r in range(BLK):
            for k in range(ACCW // LN):
                mb[0, r, pl.ds(LN * k, LN)] = zero16
        @pl.loop(0, 39)
        def _(t):
            pltpu.sync_copy(mb.at[0], acc.at[pl.ds((t * NS + sid) * BLK, BLK)])
        @pl.when(sid == 0)
        def _():
            pltpu.sync_copy(mb.at[0], acc.at[pl.ds((NCHUNK - 1) * BLK, BLK)])
        plsc.subcore_barrier()

    for s in range(2):
        zero_acc()
        q = 2 * c + s
        qbase = q * NPAD

        def gather_descs(blk, b):
            return (
                pltpu.make_async_copy(
                    xpq_h.at[colx_v[blk] + qbase], gx.at[b], gxsem[b]),
                pltpu.make_async_copy(
                    ssrc_h.at[rows_v[blk] + cN], gs.at[b], gssem[b]),
            )

        for b in range(NBUF):
            for d in gather_descs(b, b):
                d.start()

        @pl.loop(0, NB, step=NBUF)
        def _(i0):
            for b in range(NBUF):
                blk = i0 + b
                for d in gather_descs(blk, b):
                    d.wait()
                @pl.when(blk >= NBUF)
                def _():
                    pltpu.make_async_copy(
                        mb.at[b], acc.at[rows_v[blk - NBUF]], scsem[b]).wait()
                for j in range(BLK):
                    e = gs[b, j] + gx[b, j, pl.ds(0, 16)]
                    e = jnp.maximum(e, 0.2 * e)
                    ex = jnp.exp(e - mv)
                    mb[b, j, pl.ds(QW, 16)] = ex
                    for k in range(2):
                        ws = jnp.full((16,), jnp.sum(ex * ohf[2 * s + k]),
                                      jnp.float32)
                        fw = gx[b, j, pl.ds(16 + 16 * k, 16)]
                        fa, fb = plsc.unpack(
                            plsc.bitcast(fw, jnp.bfloat16),
                            format=plsc.PackFormat.INTERLEAVED)
                        mb[b, j, pl.ds(k * 32, LN)] = fa * ws
                        mb[b, j, pl.ds(k * 32 + LN, LN)] = fb * ws
                pltpu.async_copy(mb.at[b], acc.at[rows_v[blk]], scsem[b],
                                 add=True)
                @pl.when(blk + NBUF < NB)
                def _():
                    for d in gather_descs(blk + NBUF, b):
                        d.start()

        for b in range(NBUF):
            pltpu.make_async_copy(
                mb.at[b], acc.at[rows_v[NB - NBUF + b]], scsem[b]).wait()
        plsc.subcore_barrier()

        def norm_chunk(g):
            r0 = g * BLK
            pltpu.sync_copy(acc.at[pl.ds(r0, BLK)], nbuf)
            for j in range(BLK):
                for k in range(2):
                    dn = plsc.load_gather(nbuf, [jsplat[j], dsplat[2 * s + k]])
                    rec = 1.0 / jnp.maximum(dn, 1e-30)
                    for r2 in range(2):
                        off = k * 32 + r2 * LN
                        obuf[j, pl.ds(off, LN)] = nbuf[j, pl.ds(off, LN)] * rec
            pltpu.sync_copy(obuf, out_h.at[pl.ds(r0, BLK), pl.ds(QW * q, QW)])

        @pl.loop(0, 39)
        def _(t):
            norm_chunk(t * NS + sid)
        @pl.when(sid == 0)
        def _():
            norm_chunk(NCHUNK - 1)
        plsc.subcore_barrier()


def _sc_aggregate(xpq, ssrc_f, smax, row3, colx3):
    mesh = plsc.VectorSubcoreMesh(core_axis_name="c", subcore_axis_name="s")
    cp = pltpu.CompilerParams(
        needs_layout_passes=False, use_tc_tiling_on_sc=False)
    run = pl.kernel(
        _sc_body,
        compiler_params=cp,
        out_type=jax.ShapeDtypeStruct((N, D_OUT), jnp.float32),
        mesh=mesh,
        scratch_types=[
            pltpu.VMEM((NB, BLK), jnp.int32),
            pltpu.VMEM((NB, BLK), jnp.int32),
            pltpu.VMEM((NBUF, BLK, QROW), jnp.float32),
            pltpu.VMEM((NBUF, BLK, 16), jnp.float32),
            pltpu.VMEM((NBUF, BLK, ACCW), jnp.float32),
            pltpu.VMEM((16,), jnp.float32),
            pltpu.VMEM((BLK, ACCW), jnp.float32),
            pltpu.VMEM((BLK, QW), jnp.float32),
            pltpu.VMEM_SHARED((N, ACCW), jnp.float32),
            [pltpu.SemaphoreType.DMA] * NBUF,
            [pltpu.SemaphoreType.DMA] * NBUF,
            [pltpu.SemaphoreType.DMA] * NBUF,
        ],
    )
    return run(xpq, ssrc_f, smax, row3, colx3)


def kernel(x, edge_indices, W, src_attn, dst_attn):
    eye = jnp.eye(H, dtype=jnp.float32)
    asrc = (src_attn[0][:, :, None] * eye[:, None, :]).reshape(D_OUT, H)
    adst = (dst_attn[0][:, :, None] * eye[:, None, :]).reshape(D_OUT, H)
    A = jnp.concatenate(
        [asrc[:, 0:4], adst[:, 0:4], asrc[:, 4:8], adst[:, 4:8]], axis=1)

    xpq, ssrc, smax = _project(x, W, A)
    xpq_f = xpq.reshape(4 * NPAD, QROW)
    ssrc_f = ssrc.reshape(2 * N, 16)

    row = edge_indices[0].astype(jnp.int32)
    col = edge_indices[1].astype(jnp.int32)
    padn = E_PAD - E
    row3 = jnp.concatenate(
        [row, jnp.zeros((padn,), jnp.int32)]).reshape(NS, NB, BLK)
    colx3 = jnp.concatenate(
        [col, jnp.full((padn,), N, jnp.int32)]).reshape(NS, NB, BLK)

    return _sc_aggregate(xpq_f, ssrc_f, smax, row3, colx3)


# speedup vs baseline: 59.3771x; 1.1389x over previous
import functools

import jax
import jax.numpy as jnp
from jax import lax
from jax.experimental import pallas as pl
from jax.experimental.pallas import tpu as pltpu
from jax.experimental.pallas import tpu_sc as plsc

N = 10000
E = 160000
D_IN = 256
H = 8
DH = 32
D_OUT = H * DH
QW = 64
QROW = 16 + QW // 2

NC = 2
NS = 16
LN = 16

BLK = 32
NB = 316
EPT = NB * BLK
E_PAD = EPT * NS
NBUF = 4
ACCW = 80
CH = 16
NCHUNK = N // CH

_ROW_BLOCK = 1000
NPAD = N + _ROW_BLOCK


def _proj_body(x_ref, w_ref, a_ref, xpq_ref, ssrc_ref, smax_ref):
    i = pl.program_id(0)
    xp = jnp.dot(x_ref[...], w_ref[...], preferred_element_type=jnp.float32)
    s = jnp.dot(xp, a_ref[...], preferred_element_type=jnp.float32)
    z = jnp.zeros((s.shape[0], 12), jnp.float32)

    xpb = jax.lax.bitcast_convert_type(
        xp.astype(jnp.bfloat16), jnp.int16).astype(jnp.int32)
    words = []
    for k in range(8):
        lo = xpb[:, 32 * k:32 * k + 16] & 0xFFFF
        hi = xpb[:, 32 * k + 16:32 * k + 32] << 16
        words.append(lo | hi)
    packed = jax.lax.bitcast_convert_type(
        jnp.concatenate(words, axis=1), jnp.float32)

    @pl.when(i <= 9)
    def _():
        for q in range(4):
            dst = jnp.concatenate([s[:, 8 * (q // 2) + 4:8 * (q // 2) + 8], z],
                                  axis=1)
            xpq_ref[q] = jnp.concatenate(
                [dst, packed[:, 32 * q:32 * (q + 1)]], axis=1)
        ssrc_ref[0] = jnp.concatenate([s[:, 0:4], z], axis=1)
        ssrc_ref[1] = jnp.concatenate([s[:, 8:12], z], axis=1)

    @pl.when(i == 10)
    def _():
        dummy = jnp.concatenate(
            [jnp.full((_ROW_BLOCK, 16), -1e30, jnp.float32),
             jnp.zeros((_ROW_BLOCK, QW // 2), jnp.float32)], axis=1)
        for q in range(4):
            xpq_ref[q] = dummy

    bm = jnp.max(s, axis=0, keepdims=True)
    @pl.when(i == 0)
    def _():
        smax_ref[...] = jnp.full((1, 16), -1e30, jnp.float32)
    smax_ref[...] = jnp.maximum(smax_ref[...], bm)


def _project(x, W, A):
    return pl.pallas_call(
        _proj_body,
        grid=(11,),
        in_specs=[
            pl.BlockSpec((_ROW_BLOCK, D_IN), lambda i: (jnp.minimum(i, 9), 0)),
            pl.BlockSpec((D_IN, D_OUT), lambda i: (0, 0)),
            pl.BlockSpec((D_OUT, 16), lambda i: (0, 0)),
        ],
        out_specs=[
            pl.BlockSpec((4, _ROW_BLOCK, QROW), lambda i: (0, i, 0)),
            pl.BlockSpec((2, _ROW_BLOCK, 16),
                         lambda i: (0, jnp.minimum(i, 9), 0)),
            pl.BlockSpec((1, 16), lambda i: (0, 0)),
        ],
        out_shape=[
            jax.ShapeDtypeStruct((4, NPAD, QROW), jnp.float32),
            jax.ShapeDtypeStruct((2, N, 16), jnp.float32),
            jax.ShapeDtypeStruct((1, 16), jnp.float32),
        ],
    )(x, W, A)


def _sc_body(xpq_h, ssrc_h, smax_h, row3_h, colx3_h, out_h,
             rows_v, colx_v, gx, gs, mb, smax_v, nbuf, obuf,
             acc, gxsem, gssem, scsem):
    c = lax.axis_index("c")
    sid = lax.axis_index("s")
    i16 = lax.iota(jnp.int32, 16)
    cN = c * N

    pltpu.sync_copy(row3_h.at[sid], rows_v)
    pltpu.sync_copy(colx3_h.at[sid], colx_v)
    pltpu.sync_copy(smax_h.at[0], smax_v)

    ms = plsc.load_gather(smax_v, [i16 + 8 * c])
    md = plsc.load_gather(smax_v, [jnp.minimum(i16 + (8 * c + 4), 15)])
    mv = jnp.where(i16 < 4, jnp.maximum(ms + md, 0.0), 1e30)

    zero16 = jnp.zeros((16,), jnp.float32)
    ohf = [jnp.where(i16 == h, 1.0, 0.0).astype(jnp.float32) for h in range(4)]
    jsplat = [jnp.full((16,), j, jnp.int32) for j in range(CH)]
    dsplat = [jnp.full((16,), QW + h, jnp.int32) for h in range(4)]

    def zero_acc():
        for r in range(CH):
            for k in range(ACCW // LN):
                mb[0, r, pl.ds(LN * k, LN)] = zero16
        zsrc = mb.at[0, pl.ds(0, CH)]
        @pl.loop(0, 39)
        def _(t):
            pltpu.sync_copy(zsrc, acc.at[pl.ds((t * NS + sid) * CH, CH)])
        @pl.when(sid == 0)
        def _():
            pltpu.sync_copy(zsrc, acc.at[pl.ds((NCHUNK - 1) * CH, CH)])
        plsc.subcore_barrier()

    for s in range(2):
        zero_acc()
        q = 2 * c + s
        qbase = q * NPAD

        xpq_c = xpq_h.at[pl.ds(qbase, NPAD)]
        ssrc_c = ssrc_h.at[pl.ds(cN, N)]

        def gather_descs(blk, b):
            return (
                pltpu.make_async_copy(
                    xpq_c.at[colx_v.at[blk]], gx.at[b], gxsem[b]),
                pltpu.make_async_copy(
                    ssrc_c.at[rows_v.at[blk]], gs.at[b], gssem[b]),
            )

        for b in range(NBUF):
            for d in gather_descs(b, b):
                d.start()

        @pl.loop(0, NB, step=NBUF)
        def _(i0):
            for b in range(NBUF):
                blk = i0 + b
                for d in gather_descs(blk, b):
                    d.wait()
                @pl.when(blk >= NBUF)
                def _():
                    pltpu.make_async_copy(
                        mb.at[b], acc.at[rows_v.at[blk - NBUF]],
                        scsem[b]).wait()
                for j in range(BLK):
                    e = gs[b, j] + gx[b, j, pl.ds(0, 16)]
                    e = jnp.maximum(e, 0.2 * e)
                    ex = jnp.exp(e - mv)
                    mb[b, j, pl.ds(QW, 16)] = ex
                    for k in range(2):
                        ws = jnp.full((16,), jnp.sum(ex * ohf[2 * s + k]),
                                      jnp.float32)
                        fw = gx[b, j, pl.ds(16 + 16 * k, 16)]
                        fa, fb = plsc.unpack(
                            plsc.bitcast(fw, jnp.bfloat16),
                            format=plsc.PackFormat.INTERLEAVED)
                        mb[b, j, pl.ds(k * 32, LN)] = fa * ws
                        mb[b, j, pl.ds(k * 32 + LN, LN)] = fb * ws
                pltpu.async_copy(mb.at[b], acc.at[rows_v.at[blk]], scsem[b],
                                 add=True)
                @pl.when(blk + NBUF < NB)
                def _():
                    for d in gather_descs(blk + NBUF, b):
                        d.start()

        for b in range(NBUF):
            pltpu.make_async_copy(
                mb.at[b], acc.at[rows_v.at[NB - NBUF + b]], scsem[b]).wait()
        plsc.subcore_barrier()

        def norm_chunk(g):
            r0 = g * CH
            pltpu.sync_copy(acc.at[pl.ds(r0, CH)], nbuf)
            for j in range(CH):
                for k in range(2):
                    dn = plsc.load_gather(nbuf, [jsplat[j], dsplat[2 * s + k]])
                    rec = 1.0 / jnp.maximum(dn, 1e-30)
                    for r2 in range(2):
                        off = k * 32 + r2 * LN
                        obuf[j, pl.ds(off, LN)] = nbuf[j, pl.ds(off, LN)] * rec
            pltpu.sync_copy(obuf, out_h.at[pl.ds(r0, CH), pl.ds(QW * q, QW)])

        @pl.loop(0, 39)
        def _(t):
            norm_chunk(t * NS + sid)
        @pl.when(sid == 0)
        def _():
            norm_chunk(NCHUNK - 1)
        plsc.subcore_barrier()


def _sc_aggregate(xpq, ssrc_f, smax, row3, colx3):
    mesh = plsc.VectorSubcoreMesh(core_axis_name="c", subcore_axis_name="s")
    cp = pltpu.CompilerParams(
        needs_layout_passes=False, use_tc_tiling_on_sc=False)
    run = pl.kernel(
        _sc_body,
        compiler_params=cp,
        out_type=jax.ShapeDtypeStruct((N, D_OUT), jnp.float32),
        mesh=mesh,
        scratch_types=[
            pltpu.VMEM((NB, BLK), jnp.int32),
            pltpu.VMEM((NB, BLK), jnp.int32),
            pltpu.VMEM((NBUF, BLK, QROW), jnp.float32),
            pltpu.VMEM((NBUF, BLK, 16), jnp.float32),
            pltpu.VMEM((NBUF, BLK, ACCW), jnp.float32),
            pltpu.VMEM((16,), jnp.float32),
            pltpu.VMEM((CH, ACCW), jnp.float32),
            pltpu.VMEM((CH, QW), jnp.float32),
            pltpu.VMEM_SHARED((N, ACCW), jnp.float32),
            [pltpu.SemaphoreType.DMA] * NBUF,
            [pltpu.SemaphoreType.DMA] * NBUF,
            [pltpu.SemaphoreType.DMA] * NBUF,
        ],
    )
    return run(xpq, ssrc_f, smax, row3, colx3)


def kernel(x, edge_indices, W, src_attn, dst_attn):
    eye = jnp.eye(H, dtype=jnp.float32)
    asrc = (src_attn[0][:, :, None] * eye[:, None, :]).reshape(D_OUT, H)
    adst = (dst_attn[0][:, :, None] * eye[:, None, :]).reshape(D_OUT, H)
    A = jnp.concatenate(
        [asrc[:, 0:4], adst[:, 0:4], asrc[:, 4:8], adst[:, 4:8]], axis=1)

    xpq, ssrc, smax = _project(x, W, A)
    xpq_f = xpq.reshape(4 * NPAD, QROW)
    ssrc_f = ssrc.reshape(2 * N, 16)

    row = edge_indices[0].astype(jnp.int32)
    col = edge_indices[1].astype(jnp.int32)
    padn = E_PAD - E
    row3 = jnp.concatenate(
        [row, jnp.zeros((padn,), jnp.int32)]).reshape(NS, NB, BLK)
    colx3 = jnp.concatenate(
        [col, jnp.full((padn,), N, jnp.int32)]).reshape(NS, NB, BLK)

    return _sc_aggregate(xpq_f, ssrc_f, smax, row3, colx3)
